# Initial kernel scaffold; baseline (speedup 1.0000x reference)
import jax
import jax.numpy as jnp
from jax.experimental import pallas as pl


def kernel(x, edge_index, W_in, b_in, W_nor, b_nor, W_abnor, b_abnor, W_att, b_att, v_att, W_upd, b_upd, W_cls, b_cls):
    raise NotImplementedError("write your pallas kernel here")


# speedup vs baseline: 16.5228x; 16.5228x over previous
import functools

import jax
import jax.numpy as jnp
from jax import lax
from jax.experimental import pallas as pl
from jax.experimental.pallas import tpu as pltpu
from jax.experimental.pallas import tpu_sc as plsc

_N = 10000
_D = 128
_H = 128
_HALF = 64
_OUT = 2
_NEG = 0.01

_NPAD = 10240
_RB = 640
_NBLK = _NPAD // _RB
_NCORE = 2
_NSUB = 16
_NW = _NCORE * _NSUB
_CH = 128
_CPW = 80
_EPAD = _NW * _CPW * _CH

_f32 = jnp.float32

_vmesh = plsc.VectorSubcoreMesh(core_axis_name="core", subcore_axis_name="subcore")
_sc_params = pltpu.CompilerParams(needs_layout_passes=False)


def _hist_body(col_hbm, out_hbm, col_v, hist_v):
    w = lax.axis_index("core") * _NSUB + lax.axis_index("subcore")
    pltpu.sync_copy(col_hbm.at[w], col_v)

    @pl.loop(0, _NPAD // 16)
    def _zero(i):
        hist_v[pl.ds(i * 16, 16)] = jnp.zeros((16,), _f32)

    ones = jnp.ones((16,), _f32)

    @pl.loop(0, _CPW)
    def _chunk(j):
        @pl.loop(0, _CH // 16)
        def _vec(c):
            iv = col_v[j, pl.ds(c * 16, 16)]
            plsc.addupdate_scatter(hist_v, [iv], ones)

    pltpu.sync_copy(hist_v, out_hbm.at[w])


@jax.jit
def _hist_call(col_p):
    return pl.kernel(
        _hist_body,
        out_type=jax.ShapeDtypeStruct((_NW, _NPAD), _f32),
        mesh=_vmesh,
        compiler_params=_sc_params,
        scratch_types=[
            pltpu.VMEM((_CPW, _CH), jnp.int32),
            pltpu.VMEM((_NPAD,), _f32),
        ],
    )(col_p)


def _agg_body(p_hbm, ridx_hbm, cidx_hbm, q_hbm, ridx_v, cidx_v, rows_v, q_sh, sem):
    cid = lax.axis_index("core")
    sid = lax.axis_index("subcore")
    w = cid * _NSUB + sid

    @pl.loop(0, _CH)
    def _zr(i):
        @pl.loop(0, _D // 16)
        def _zc(c):
            rows_v[i, pl.ds(c * 16, 16)] = jnp.zeros((16,), _f32)

    @pl.loop(0, _RB // _CH)
    def _zs(k):
        pltpu.sync_copy(rows_v, q_sh.at[pl.ds(sid * _RB + k * _CH, _CH)])

    plsc.subcore_barrier()

    pltpu.sync_copy(ridx_hbm.at[w], ridx_v)
    pltpu.sync_copy(cidx_hbm.at[w], cidx_v)

    @pl.loop(0, _CPW)
    def _edge(j):
        pltpu.async_copy(p_hbm.at[ridx_v.at[j]], rows_v, sem).wait()
        pltpu.sync_copy(rows_v, q_sh.at[cidx_v.at[j]], add=True)

    plsc.subcore_barrier()

    pltpu.sync_copy(q_sh.at[pl.ds(sid * _RB, _RB)],
                    q_hbm.at[cid, pl.ds(sid * _RB, _RB)])


@jax.jit
def _agg_call(p, ridx, cidx):
    return pl.kernel(
        _agg_body,
        out_type=jax.ShapeDtypeStruct((_NCORE, _NPAD, _D), _f32),
        mesh=_vmesh,
        scratch_types=[
            pltpu.VMEM((_CPW, _CH), jnp.int32),
            pltpu.VMEM((_CPW, _CH), jnp.int32),
            pltpu.VMEM((_CH, _D), _f32),
            pltpu.VMEM_SHARED((_NPAD, _D), _f32),
            pltpu.SemaphoreType.DMA,
        ],
    )(p, ridx, cidx)


def _leaky(v):
    return jnp.where(v > 0, v, v * _NEG)


def _dense1_body(x_ref, degt_ref, wi, bi, wn, bn, wa, ba, wt, bt, va, p_ref):
    xb = x_ref[...]
    h = _leaky(jnp.dot(xb, wi[...], preferred_element_type=_f32) + bi[...])
    xn = jnp.dot(h[:, :_HALF], wn[...], preferred_element_type=_f32) + bn[...]
    xa = jnp.dot(h[:, _HALF:], wa[...], preferred_element_type=_f32) + ba[...]
    t = jnp.tanh(jnp.dot(xn + xa, wt[...], preferred_element_type=_f32) + bt[...])
    a = jax.nn.sigmoid(jnp.sum(t * va[...], axis=1, keepdims=True))
    m = a * xn + (1.0 - a) * xa
    deg = jnp.sum(degt_ref[...], axis=1, keepdims=True) + 1.0
    p_ref[...] = lax.rsqrt(deg) * m


@jax.jit
def _dense1_call(xp, degt, wi, bi, wn, bn, wa, ba, wt, bt, va):
    full = lambda s: pl.BlockSpec(s, lambda i: (0,) * len(s))
    return pl.pallas_call(
        _dense1_body,
        grid=(_NBLK,),
        in_specs=[
            pl.BlockSpec((_RB, _D), lambda i: (i, 0)),
            pl.BlockSpec((_RB, _NW), lambda i: (i, 0)),
            full((_D, _H)), full((1, _H)),
            full((_HALF, _H)), full((1, _H)),
            full((_HALF, _H)), full((1, _H)),
            full((_H, _H)), full((1, _H)),
            full((1, _H)),
        ],
        out_specs=pl.BlockSpec((_RB, _D), lambda i: (i, 0)),
        out_shape=jax.ShapeDtypeStruct((_NPAD, _D), _f32),
    )(xp, degt, wi, bi, wn, bn, wa, ba, wt, bt, va)


def _dense2_body(q_ref, p_ref, degt_ref, wu, bu, wc, bc, o_ref):
    q = q_ref[0] + q_ref[1]
    deg = jnp.sum(degt_ref[...], axis=1, keepdims=True) + 1.0
    aggr = lax.rsqrt(deg) * (q + p_ref[...])
    h2 = _leaky(jnp.dot(aggr, wu[...], preferred_element_type=_f32) + bu[...])
    o_ref[...] = jnp.dot(h2, wc[...], preferred_element_type=_f32) + bc[...]


@jax.jit
def _dense2_call(q, p, degt, wu, bu, wc, bc):
    full = lambda s: pl.BlockSpec(s, lambda i: (0,) * len(s))
    return pl.pallas_call(
        _dense2_body,
        grid=(_NBLK,),
        in_specs=[
            pl.BlockSpec((_NCORE, _RB, _D), lambda i: (0, i, 0)),
            pl.BlockSpec((_RB, _D), lambda i: (i, 0)),
            pl.BlockSpec((_RB, _NW), lambda i: (i, 0)),
            full((_H, _H)), full((1, _H)),
            full((_H, _OUT)), full((1, _OUT)),
        ],
        out_specs=pl.BlockSpec((_RB, _OUT), lambda i: (i, 0)),
        out_shape=jax.ShapeDtypeStruct((_NPAD, _OUT), _f32),
    )(q, p, degt, wu, bu, wc, bc)


def kernel(x, edge_index, W_in, b_in, W_nor, b_nor, W_abnor, b_abnor,
           W_att, b_att, v_att, W_upd, b_upd, W_cls, b_cls):
    row, col = edge_index[0], edge_index[1]
    e = row.shape[0]
    pad = _EPAD - e
    row_p = jnp.concatenate([row, jnp.zeros((pad,), row.dtype)]).reshape(_NW, _CPW, _CH)
    col_p = jnp.concatenate([col, jnp.full((pad,), _NPAD - 1, col.dtype)]).reshape(_NW, _CPW, _CH)
    xp = jnp.pad(x, ((0, _NPAD - _N), (0, 0)))

    deg_part = _hist_call(col_p)
    degt = deg_part.T
    p = _dense1_call(xp, degt, W_in, b_in.reshape(1, -1), W_nor, b_nor.reshape(1, -1),
                     W_abnor, b_abnor.reshape(1, -1), W_att, b_att.reshape(1, -1),
                     v_att.reshape(1, -1))
    q = _agg_call(p, row_p, col_p)
    out = _dense2_call(q, p, degt, W_upd, b_upd.reshape(1, -1),
                       W_cls, b_cls.reshape(1, -1))
    return out[:_N]
